# baseline (speedup 1.0000x reference)
import functools

import jax
import jax.numpy as jnp
from jax import lax
from jax.experimental import pallas as pl
from jax.experimental.pallas import tpu as pltpu
from jax.experimental.pallas import tpu_sc as plsc

_N = 16
_C = 68
_H = 128
_W = 128
_HW = _H * _W
_K = 9
_L = 16
_NW = 32
_ROWS = _N * _C
_RPW = _ROWS // _NW
_NACC = 8
_Q = _HW // _NACC
_QSH = _Q.bit_length() - 1
_BPB = _Q // _L
_BIG = 1 << 30


def _splat(x):
    return jnp.full((_L,), x)


def _row_topk_coord(buf, j, outbuf):
    iota = lax.iota(jnp.int32, _L)
    viota16 = iota * 16
    ninf_v = _splat(jnp.float32(-jnp.inf))
    zero_i = jnp.zeros((_L,), jnp.int32)

    _UNROLL = 1

    def fold_body(b, carry):
        avs, ais = carry
        avs, ais = list(avs), list(ais)
        off0 = b * 16
        idxv = iota + off0
        for t in range(_NACC):
            v = buf[pl.ds(t * _Q + off0, _L)]
            gt = v > avs[t]
            avs[t] = jnp.where(gt, v, avs[t])
            ais[t] = jnp.where(gt, idxv, ais[t])
        return tuple(avs), tuple(ais)

    avs, ais = lax.fori_loop(
        0, _BPB // _UNROLL, fold_body,
        ((ninf_v,) * _NACC, (zero_i,) * _NACC))

    _BIGF = jnp.float32(1 << 24)
    avs = list(avs)
    ais = list(ais)
    s_vec, i_vec = ninf_v, zero_i

    def _maxtree(vs):
        vs = list(vs)
        while len(vs) > 1:
            vs = [jnp.maximum(a, b) for a, b in zip(vs[::2], vs[1::2])]
        return vs[0]

    m = _splat(jnp.max(_maxtree(avs)))
    for k in range(_K):
        cands = [jnp.where(avs[t] == m,
                           (ais[t] + t * _Q if t else ais[t])
                           .astype(jnp.float32), _BIGF)
                 for t in range(_NACC)]
        while len(cands) > 1:
            cands = [jnp.minimum(a, b) for a, b in zip(cands[::2], cands[1::2])]
        wi = _splat(jnp.min(cands[0])).astype(jnp.int32)
        s_vec = jnp.where(iota == k, m, s_vec)
        i_vec = jnp.where(iota == k, wi, i_vec)
        if k == _K - 1:
            break
        plsc.store_scatter(buf, [wi], ninf_v, mask=iota == 0)
        is_lane = iota == (wi & 15)
        tq = wi >> _QSH
        hits = [is_lane & (tq == t) for t in range(_NACC)]
        mm_wo = _maxtree([jnp.where(hits[t], -jnp.inf, avs[t])
                          for t in range(_NACC)])
        m_wo = _splat(jnp.max(mm_wo))
        base = (wi & 15) + (wi & ((_NACC - 1) << _QSH))
        idxa = base + viota16
        idxb = idxa + 256
        rva, ria = ninf_v, zero_i
        rvb, rib = ninf_v, zero_i
        for _ in range(_BPB // _L // 2):
            ga = plsc.load_gather(buf, [idxa])
            gb = plsc.load_gather(buf, [idxb])
            gta = ga > rva
            gtb = gb > rvb
            rva = jnp.where(gta, ga, rva)
            ria = jnp.where(gta, idxa, ria)
            rvb = jnp.where(gtb, gb, rvb)
            rib = jnp.where(gtb, idxb, rib)
            idxa = idxa + 512
            idxb = idxb + 512
        takeb = (rvb > rva) | ((rvb == rva) & (rib < ria))
        rv = jnp.where(takeb, rvb, rva)
        ri = jnp.where(takeb, rib, ria)
        m2 = _splat(jnp.max(rv))
        rif = ri.astype(jnp.float32)
        w2 = _splat(jnp.min(jnp.where(rv == m2, rif, _BIGF))).astype(jnp.int32)
        w2 = w2 & (_Q - 1)
        for t in range(_NACC):
            avs[t] = jnp.where(hits[t], m2, avs[t])
            ais[t] = jnp.where(hits[t], w2, ais[t])
        m = jnp.maximum(m_wo, m2)

    smax = _splat(jnp.max(s_vec))
    w = jnp.where(iota < _K, jnp.exp(s_vec - smax), jnp.float32(0.0))
    den = jnp.sum(w)
    xf = (i_vec & (_W - 1)).astype(jnp.float32)
    yf = (i_vec >> 7).astype(jnp.float32)
    nx = jnp.sum(w * xf)
    ny = jnp.sum(w * yf)
    ox = _splat(nx) * 4.0 / _splat(den)
    oy = _splat(ny) * 4.0 / _splat(den)
    res = jnp.where(iota == 0, ox, jnp.where(iota == 1, oy, jnp.float32(0.0)))
    outbuf[pl.ds(j * _L, _L)] = res


def _make_sc_call():
    mesh = plsc.VectorSubcoreMesh(core_axis_name="c", subcore_axis_name="s")

    @functools.partial(
        pl.kernel,
        out_type=jax.ShapeDtypeStruct((_NW * _RPW * _L,), jnp.float32),
        mesh=mesh,
        scratch_types=[
            pltpu.VMEM((_HW,), jnp.float32),
            pltpu.VMEM((_HW,), jnp.float32),
            pltpu.VMEM((_RPW * _L,), jnp.float32),
            pltpu.SemaphoreType.DMA,
            pltpu.SemaphoreType.DMA,
        ],
        compiler_params=pltpu.CompilerParams(needs_layout_passes=False),
    )
    def sc_kernel(x_hbm, out_hbm, buf0, buf1, outbuf, sem0, sem1):
        wid = lax.axis_index("s") * 2 + lax.axis_index("c")

        def hslice(j):
            r = wid * _RPW + j
            n = r // _C
            c = r % _C
            hrow = n * (2 * _C) + _C + c
            return pl.ds(hrow * _HW, _HW)

        pltpu.async_copy(x_hbm.at[hslice(0)], buf0, sem0)

        def pair_body(p, carry):
            j0 = p * 2
            pltpu.async_copy(x_hbm.at[hslice(j0 + 1)], buf1, sem1)
            pltpu.make_async_copy(x_hbm.at[hslice(j0)], buf0, sem0).wait()
            _row_topk_coord(buf0, j0, outbuf)

            @pl.when(p < _RPW // 2 - 1)
            def _():
                pltpu.async_copy(x_hbm.at[hslice(j0 + 2)], buf0, sem0)

            pltpu.make_async_copy(x_hbm.at[hslice(j0 + 1)], buf1, sem1).wait()
            _row_topk_coord(buf1, j0 + 1, outbuf)
            return carry

        lax.fori_loop(0, _RPW // 2, pair_body, 0)
        pltpu.sync_copy(outbuf, out_hbm.at[pl.ds(wid * _RPW * _L, _RPW * _L)])

    return sc_kernel


_sc_call = _make_sc_call()


@jax.jit
def kernel(input):
    x = input.reshape(_N * 2 * _C * _HW)
    out = _sc_call(x)
    out = out.reshape(_ROWS, _L)[:, :2]
    return out.reshape(_N, _C, 2)


# speedup vs baseline: 1.0024x; 1.0024x over previous
import functools

import jax
import jax.numpy as jnp
from jax import lax
from jax.experimental import pallas as pl
from jax.experimental.pallas import tpu as pltpu
from jax.experimental.pallas import tpu_sc as plsc

_N = 16
_C = 68
_H = 128
_W = 128
_HW = _H * _W
_K = 9
_L = 16
_NW = 32
_ROWS = _N * _C
_RPW = _ROWS // _NW
_NACC = 8
_Q = _HW // _NACC
_QSH = _Q.bit_length() - 1
_BPB = _Q // _L


def _splat(x):
    return jnp.full((_L,), x)


def _row_topk_coord(buf, j, outbuf):
    iota = lax.iota(jnp.int32, _L)
    viota16 = iota * 16
    ninf_v = _splat(jnp.float32(-jnp.inf))
    zero_i = jnp.zeros((_L,), jnp.int32)


    def fold_body(b, carry):
        avs, ais = carry
        avs, ais = list(avs), list(ais)
        off0 = b * 16
        idxv = iota + off0
        for t in range(_NACC):
            v = buf[pl.ds(t * _Q + off0, _L)]
            gt = v > avs[t]
            avs[t] = jnp.where(gt, v, avs[t])
            ais[t] = jnp.where(gt, idxv, ais[t])
        return tuple(avs), tuple(ais)

    avs, ais = lax.fori_loop(
        0, _BPB, fold_body,
        ((ninf_v,) * _NACC, (zero_i,) * _NACC))

    _BIGF = jnp.float32(1 << 24)
    avs = list(avs)
    ais = list(ais)
    s_vec, i_vec = ninf_v, zero_i

    def _maxtree(vs):
        vs = list(vs)
        while len(vs) > 1:
            vs = [jnp.maximum(a, b) for a, b in zip(vs[::2], vs[1::2])]
        return vs[0]

    m = _splat(jnp.max(_maxtree(avs)))
    for k in range(_K):
        cands = [jnp.where(avs[t] == m,
                           (ais[t] + t * _Q if t else ais[t])
                           .astype(jnp.float32), _BIGF)
                 for t in range(_NACC)]
        while len(cands) > 1:
            cands = [jnp.minimum(a, b) for a, b in zip(cands[::2], cands[1::2])]
        wi = _splat(jnp.min(cands[0])).astype(jnp.int32)
        s_vec = jnp.where(iota == k, m, s_vec)
        i_vec = jnp.where(iota == k, wi, i_vec)
        if k == _K - 1:
            break
        plsc.store_scatter(buf, [wi], ninf_v, mask=iota == 0)
        is_lane = iota == (wi & 15)
        tq = wi >> _QSH
        hits = [is_lane & (tq == t) for t in range(_NACC)]
        mm_wo = _maxtree([jnp.where(hits[t], -jnp.inf, avs[t])
                          for t in range(_NACC)])
        m_wo = _splat(jnp.max(mm_wo))
        base = (wi & 15) + (wi & ((_NACC - 1) << _QSH))
        idxa = base + viota16
        idxb = idxa + 256
        rva, ria = ninf_v, zero_i
        rvb, rib = ninf_v, zero_i
        for _ in range(_BPB // _L // 2):
            ga = plsc.load_gather(buf, [idxa])
            gb = plsc.load_gather(buf, [idxb])
            gta = ga > rva
            gtb = gb > rvb
            rva = jnp.where(gta, ga, rva)
            ria = jnp.where(gta, idxa, ria)
            rvb = jnp.where(gtb, gb, rvb)
            rib = jnp.where(gtb, idxb, rib)
            idxa = idxa + 512
            idxb = idxb + 512
        takeb = (rvb > rva) | ((rvb == rva) & (rib < ria))
        rv = jnp.where(takeb, rvb, rva)
        ri = jnp.where(takeb, rib, ria)
        m2 = _splat(jnp.max(rv))
        rif = ri.astype(jnp.float32)
        w2 = _splat(jnp.min(jnp.where(rv == m2, rif, _BIGF))).astype(jnp.int32)
        w2 = w2 & (_Q - 1)
        for t in range(_NACC):
            avs[t] = jnp.where(hits[t], m2, avs[t])
            ais[t] = jnp.where(hits[t], w2, ais[t])
        m = jnp.maximum(m_wo, m2)

    smax = _splat(jnp.max(s_vec))
    w = jnp.where(iota < _K, jnp.exp(s_vec - smax), jnp.float32(0.0))
    den = jnp.sum(w)
    xf = (i_vec & (_W - 1)).astype(jnp.float32)
    yf = (i_vec >> 7).astype(jnp.float32)
    nx = jnp.sum(w * xf)
    ny = jnp.sum(w * yf)
    ox = _splat(nx) * 4.0 / _splat(den)
    oy = _splat(ny) * 4.0 / _splat(den)
    res = jnp.where(iota == 0, ox, jnp.where(iota == 1, oy, jnp.float32(0.0)))
    outbuf[pl.ds(j * _L, _L)] = res


def _make_sc_call():
    mesh = plsc.VectorSubcoreMesh(core_axis_name="c", subcore_axis_name="s")

    @functools.partial(
        pl.kernel,
        out_type=jax.ShapeDtypeStruct((_NW * _RPW * _L,), jnp.float32),
        mesh=mesh,
        scratch_types=[
            pltpu.VMEM((_HW,), jnp.float32),
            pltpu.VMEM((_HW,), jnp.float32),
            pltpu.VMEM((_RPW * _L,), jnp.float32),
            pltpu.SemaphoreType.DMA,
            pltpu.SemaphoreType.DMA,
        ],
        compiler_params=pltpu.CompilerParams(needs_layout_passes=False),
    )
    def sc_kernel(x_hbm, out_hbm, buf0, buf1, outbuf, sem0, sem1):
        wid = lax.axis_index("s") * 2 + lax.axis_index("c")

        def hslice(j):
            r = wid * _RPW + j
            n = r // _C
            c = r % _C
            hrow = n * (2 * _C) + _C + c
            return pl.ds(hrow * _HW, _HW)

        pltpu.async_copy(x_hbm.at[hslice(0)], buf0, sem0)

        def pair_body(p, carry):
            j0 = p * 2
            pltpu.async_copy(x_hbm.at[hslice(j0 + 1)], buf1, sem1)
            pltpu.make_async_copy(x_hbm.at[hslice(j0)], buf0, sem0).wait()
            _row_topk_coord(buf0, j0, outbuf)

            @pl.when(p < _RPW // 2 - 1)
            def _():
                pltpu.async_copy(x_hbm.at[hslice(j0 + 2)], buf0, sem0)

            pltpu.make_async_copy(x_hbm.at[hslice(j0 + 1)], buf1, sem1).wait()
            _row_topk_coord(buf1, j0 + 1, outbuf)
            return carry

        lax.fori_loop(0, _RPW // 2, pair_body, 0)
        pltpu.sync_copy(outbuf, out_hbm.at[pl.ds(wid * _RPW * _L, _RPW * _L)])

    return sc_kernel


_sc_call = _make_sc_call()


@jax.jit
def kernel(input):
    x = input.reshape(_N * 2 * _C * _HW)
    out = _sc_call(x)
    out = out.reshape(_ROWS, _L)[:, :2]
    return out.reshape(_N, _C, 2)
